# baseline (speedup 1.0000x reference)
import functools

import jax
import jax.numpy as jnp
from jax.experimental import pallas as pl
from jax.experimental.pallas import tpu as pltpu


def _stats_body(s0_ref, x_ref, rel_ref, c_ref, invc_ref, ms_ref, w_ref,
                b_ref, a_ref, c2_ref, *, R, N, G, W):
    i = pl.program_id(0)
    dot = functools.partial(jnp.dot, preferred_element_type=jnp.float32,
                            precision=jax.lax.Precision.HIGHEST)

    @pl.when(i == 0)
    def _():
        a_ref[...] = jnp.zeros_like(a_ref)
        c2_ref[...] = jnp.zeros_like(c2_ref)

    s0 = pl.multiple_of(s0_ref[i], 16)
    iota_col = jax.lax.broadcasted_iota(
        jnp.int32, (W, 1), 0).astype(jnp.float32)
    oh = (rel_ref[0] == iota_col).astype(jnp.float32)
    rg_col = i * R + jax.lax.broadcasted_iota(jnp.int32, (R, 1), 0)
    x = jnp.where(rg_col < N, x_ref[...], 0.0)
    a_ref[pl.ds(s0, W), :] += dot(oh, x)
    c2_ref[pl.ds(s0, W), :] += dot(oh, x * x)

    @pl.when(i == G - 1)
    def _():
        s = a_ref[...]
        mean = s * invc_ref[...]
        msm = mean * ms_ref[...]
        varsum = jnp.maximum(
            c2_ref[...] - 2.0 * msm * s + c_ref[...] * msm * msm, 0.0)
        a = w_ref[...] * jax.lax.rsqrt(varsum * invc_ref[...] + 1e-6)
        a_ref[...] = a
        c2_ref[...] = b_ref[...] - a * msm


def _apply_body(s0_ref, x_ref, rel_ref, a_ref, c2_ref, o_ref, *, W):
    i = pl.program_id(0)
    dot = functools.partial(jnp.dot, preferred_element_type=jnp.float32,
                            precision=jax.lax.Precision.HIGHEST)
    s0 = pl.multiple_of(s0_ref[i], 16)
    iota_row = jax.lax.broadcasted_iota(
        jnp.int32, (1, W), 1).astype(jnp.float32)
    oh = (rel_ref[0] == iota_row).astype(jnp.float32)
    ar = dot(oh, a_ref[pl.ds(s0, W), :])
    cr = dot(oh, c2_ref[pl.ds(s0, W), :])
    o_ref[...] = ar * x_ref[...] + cr


@jax.jit
def kernel(tensor, nodes_per_img, weight, bias, mean_scale):
    N, D = tensor.shape
    B = nodes_per_img.shape[0]
    R = 2048
    G = pl.cdiv(N, R)
    Bp = 320
    W = 80

    counts = nodes_per_img.astype(jnp.float32)
    sizes = nodes_per_img.astype(jnp.int32)
    hi = jnp.cumsum(sizes)
    c_col = jnp.zeros((Bp, 1), jnp.float32).at[:B, 0].set(counts)
    invc_col = 1.0 / (c_col + 1e-6)

    blk_start = jnp.arange(G, dtype=jnp.int32) * R
    first_seg = jnp.searchsorted(hi, blk_start, side="right").astype(jnp.int32)
    s0 = jnp.minimum((first_seg // 16) * 16, Bp - W)
    segid = jnp.repeat(jnp.arange(B, dtype=jnp.int32), sizes,
                       total_repeat_length=N)
    s0_rep = jnp.repeat(s0, R, total_repeat_length=G * R)[:N]
    rel = jnp.full((G * R,), 255, jnp.int32).at[:N].set(segid - s0_rep)
    rel_f = rel.astype(jnp.float32)
    rel_row = rel_f.reshape(G, 1, R)
    rel_col = rel_f.reshape(G, R, 1)

    def const(shape):
        return pl.BlockSpec(shape, lambda i, s0r: (0,) * len(shape))

    a_mat, c_mat = pl.pallas_call(
        functools.partial(_stats_body, R=R, N=N, G=G, W=W),
        grid_spec=pltpu.PrefetchScalarGridSpec(
            num_scalar_prefetch=1,
            grid=(G,),
            in_specs=[
                pl.BlockSpec((R, D), lambda i, s0r: (i, 0)),
                pl.BlockSpec((1, 1, R), lambda i, s0r: (i, 0, 0)),
                const((Bp, 1)), const((Bp, 1)),
                const((1, D)), const((1, D)), const((1, D)),
            ],
            out_specs=[const((Bp, D)), const((Bp, D))],
        ),
        out_shape=[jax.ShapeDtypeStruct((Bp, D), jnp.float32),
                   jax.ShapeDtypeStruct((Bp, D), jnp.float32)],
    )(
        s0, tensor, rel_row, c_col, invc_col,
        mean_scale.reshape(1, D), weight.reshape(1, D), bias.reshape(1, D),
    )

    out = pl.pallas_call(
        functools.partial(_apply_body, W=W),
        grid_spec=pltpu.PrefetchScalarGridSpec(
            num_scalar_prefetch=1,
            grid=(G,),
            in_specs=[
                pl.BlockSpec((R, D), lambda i, s0r: (i, 0)),
                pl.BlockSpec((1, R, 1), lambda i, s0r: (i, 0, 0)),
                const((Bp, D)), const((Bp, D)),
            ],
            out_specs=pl.BlockSpec((R, D), lambda i, s0r: (i, 0)),
        ),
        out_shape=jax.ShapeDtypeStruct((N, D), jnp.float32),
    )(s0, tensor, rel_col, a_mat, c_mat)
    return out


# speedup vs baseline: 1.6187x; 1.6187x over previous
import functools

import jax
import jax.numpy as jnp
from jax.experimental import pallas as pl


def _stats_body(x_ref, lo_ref, hi_ref, c_ref, invc_ref, ms_ref, w_ref,
                b_ref, a_ref, c2_ref, *, R, N, G):
    i = pl.program_id(0)
    dot = functools.partial(jnp.dot, preferred_element_type=jnp.float32,
                            precision=jax.lax.Precision.HIGHEST)

    @pl.when(i == 0)
    def _():
        a_ref[...] = jnp.zeros_like(a_ref)
        c2_ref[...] = jnp.zeros_like(c2_ref)

    rg_row = i * R + jax.lax.broadcasted_iota(jnp.int32, (1, R), 1)
    oh = ((rg_row >= lo_ref[...]) & (rg_row < hi_ref[...])).astype(
        jnp.float32)
    rg_col = i * R + jax.lax.broadcasted_iota(jnp.int32, (R, 1), 0)
    x = jnp.where(rg_col < N, x_ref[...], 0.0)
    a_ref[...] += dot(oh, x)
    c2_ref[...] += dot(oh, x * x)

    @pl.when(i == G - 1)
    def _():
        s = a_ref[...]
        mean = s * invc_ref[...]
        msm = mean * ms_ref[...]
        varsum = jnp.maximum(
            c2_ref[...] - 2.0 * msm * s + c_ref[...] * msm * msm, 0.0)
        a = w_ref[...] * jax.lax.rsqrt(varsum * invc_ref[...] + 1e-6)
        a_ref[...] = a
        c2_ref[...] = b_ref[...] - a * msm


def _apply_body(x_ref, lo_ref, hi_ref, a_ref, c2_ref, o_ref, *, R):
    i = pl.program_id(0)
    dot = functools.partial(jnp.dot, preferred_element_type=jnp.float32,
                            precision=jax.lax.Precision.HIGHEST)
    rg_col = i * R + jax.lax.broadcasted_iota(jnp.int32, (R, 1), 0)
    oh = ((rg_col >= lo_ref[...]) & (rg_col < hi_ref[...])).astype(
        jnp.float32)
    ar = dot(oh, a_ref[...])
    cr = dot(oh, c2_ref[...])
    o_ref[...] = ar * x_ref[...] + cr


@jax.jit
def kernel(tensor, nodes_per_img, weight, bias, mean_scale):
    N, D = tensor.shape
    B = nodes_per_img.shape[0]
    R = 2048
    G = pl.cdiv(N, R)
    Bp = 384

    counts = nodes_per_img.astype(jnp.float32)
    sizes = nodes_per_img.astype(jnp.int32)
    hi = jnp.cumsum(sizes)
    lo = hi - sizes
    lo_p = jnp.full((Bp,), N, jnp.int32).at[:B].set(lo)
    hi_p = jnp.full((Bp,), N, jnp.int32).at[:B].set(hi)
    c_col = jnp.zeros((Bp, 1), jnp.float32).at[:B, 0].set(counts)
    invc_col = 1.0 / (c_col + 1e-6)

    def const(shape):
        return pl.BlockSpec(shape, lambda i: (0,) * len(shape))

    a_mat, c_mat = pl.pallas_call(
        functools.partial(_stats_body, R=R, N=N, G=G),
        grid=(G,),
        in_specs=[
            pl.BlockSpec((R, D), lambda i: (i, 0)),
            const((Bp, 1)), const((Bp, 1)),
            const((Bp, 1)), const((Bp, 1)),
            const((1, D)), const((1, D)), const((1, D)),
        ],
        out_specs=[const((Bp, D)), const((Bp, D))],
        out_shape=[jax.ShapeDtypeStruct((Bp, D), jnp.float32),
                   jax.ShapeDtypeStruct((Bp, D), jnp.float32)],
    )(
        tensor, lo_p.reshape(Bp, 1), hi_p.reshape(Bp, 1), c_col, invc_col,
        mean_scale.reshape(1, D), weight.reshape(1, D), bias.reshape(1, D),
    )

    out = pl.pallas_call(
        functools.partial(_apply_body, R=R),
        grid=(G,),
        in_specs=[
            pl.BlockSpec((R, D), lambda i: (i, 0)),
            const((1, Bp)), const((1, Bp)),
            const((Bp, D)), const((Bp, D)),
        ],
        out_specs=pl.BlockSpec((R, D), lambda i: (i, 0)),
        out_shape=jax.ShapeDtypeStruct((N, D), jnp.float32),
    )(tensor, lo_p.reshape(1, Bp), hi_p.reshape(1, Bp), a_mat, c_mat)
    return out


# speedup vs baseline: 4.1823x; 2.5838x over previous
import functools

import jax
import jax.numpy as jnp
from jax.experimental import pallas as pl


def _stats_body(x_ref, lo_ref, hi_ref, c_ref, invc_ref, ms_ref, w_ref,
                b_ref, a_ref, c2_ref, *, R, N, G):
    i = pl.program_id(0)
    dot = functools.partial(jnp.dot, preferred_element_type=jnp.float32)

    @pl.when(i == 0)
    def _():
        a_ref[...] = jnp.zeros_like(a_ref)
        c2_ref[...] = jnp.zeros_like(c2_ref)

    rg_row = i * R + jax.lax.broadcasted_iota(jnp.int32, (1, R), 1)
    oh = ((rg_row >= lo_ref[...]) & (rg_row < hi_ref[...])).astype(
        jnp.bfloat16)
    rg_col = i * R + jax.lax.broadcasted_iota(jnp.int32, (R, 1), 0)
    x = jnp.where(rg_col < N, x_ref[...], 0.0)
    x2 = x * x
    xh = x.astype(jnp.bfloat16)
    xl = (x - xh.astype(jnp.float32)).astype(jnp.bfloat16)
    x2h = x2.astype(jnp.bfloat16)
    x2l = (x2 - x2h.astype(jnp.float32)).astype(jnp.bfloat16)
    a_ref[...] += dot(oh, xh) + dot(oh, xl)
    c2_ref[...] += dot(oh, x2h) + dot(oh, x2l)

    @pl.when(i == G - 1)
    def _():
        s = a_ref[...]
        mean = s * invc_ref[...]
        msm = mean * ms_ref[...]
        varsum = jnp.maximum(
            c2_ref[...] - 2.0 * msm * s + c_ref[...] * msm * msm, 0.0)
        a = w_ref[...] * jax.lax.rsqrt(varsum * invc_ref[...] + 1e-6)
        a_ref[...] = a
        c2_ref[...] = b_ref[...] - a * msm


def _apply_body(x_ref, lo_ref, hi_ref, a_ref, c2_ref, o_ref, *, R):
    i = pl.program_id(0)
    dot = functools.partial(jnp.dot, preferred_element_type=jnp.float32)
    rg_col = i * R + jax.lax.broadcasted_iota(jnp.int32, (R, 1), 0)
    oh = ((rg_col >= lo_ref[...]) & (rg_col < hi_ref[...])).astype(
        jnp.bfloat16)
    a = a_ref[...]
    c = c2_ref[...]
    ah = a.astype(jnp.bfloat16)
    al = (a - ah.astype(jnp.float32)).astype(jnp.bfloat16)
    ch = c.astype(jnp.bfloat16)
    cl = (c - ch.astype(jnp.float32)).astype(jnp.bfloat16)
    ar = dot(oh, ah) + dot(oh, al)
    cr = dot(oh, ch) + dot(oh, cl)
    o_ref[...] = ar * x_ref[...] + cr


@jax.jit
def kernel(tensor, nodes_per_img, weight, bias, mean_scale):
    N, D = tensor.shape
    B = nodes_per_img.shape[0]
    R = 2048
    G = pl.cdiv(N, R)
    Bp = 384

    counts = nodes_per_img.astype(jnp.float32)
    sizes = nodes_per_img.astype(jnp.int32)
    hi = jnp.cumsum(sizes)
    lo = hi - sizes
    lo_p = jnp.full((Bp,), N, jnp.int32).at[:B].set(lo)
    hi_p = jnp.full((Bp,), N, jnp.int32).at[:B].set(hi)
    c_col = jnp.zeros((Bp, 1), jnp.float32).at[:B, 0].set(counts)
    invc_col = 1.0 / (c_col + 1e-6)

    def const(shape):
        return pl.BlockSpec(shape, lambda i: (0,) * len(shape))

    a_mat, c_mat = pl.pallas_call(
        functools.partial(_stats_body, R=R, N=N, G=G),
        grid=(G,),
        in_specs=[
            pl.BlockSpec((R, D), lambda i: (i, 0)),
            const((Bp, 1)), const((Bp, 1)),
            const((Bp, 1)), const((Bp, 1)),
            const((1, D)), const((1, D)), const((1, D)),
        ],
        out_specs=[const((Bp, D)), const((Bp, D))],
        out_shape=[jax.ShapeDtypeStruct((Bp, D), jnp.float32),
                   jax.ShapeDtypeStruct((Bp, D), jnp.float32)],
    )(
        tensor, lo_p.reshape(Bp, 1), hi_p.reshape(Bp, 1), c_col, invc_col,
        mean_scale.reshape(1, D), weight.reshape(1, D), bias.reshape(1, D),
    )

    out = pl.pallas_call(
        functools.partial(_apply_body, R=R),
        grid=(G,),
        in_specs=[
            pl.BlockSpec((R, D), lambda i: (i, 0)),
            const((1, Bp)), const((1, Bp)),
            const((Bp, D)), const((Bp, D)),
        ],
        out_specs=pl.BlockSpec((R, D), lambda i: (i, 0)),
        out_shape=jax.ShapeDtypeStruct((N, D), jnp.float32),
    )(tensor, lo_p.reshape(1, Bp), hi_p.reshape(1, Bp), a_mat, c_mat)
    return out


# speedup vs baseline: 4.7061x; 1.1253x over previous
import functools

import jax
import jax.numpy as jnp
from jax.experimental import pallas as pl


def _stats_body(x_ref, lo_ref, hi_ref, c_ref, invc_ref, ms_ref, w_ref,
                b_ref, a_ref, c2_ref, *, R, N, G):
    i = pl.program_id(0)
    dot = functools.partial(jnp.dot, preferred_element_type=jnp.float32)

    @pl.when(i == 0)
    def _():
        a_ref[...] = jnp.zeros_like(a_ref)
        c2_ref[...] = jnp.zeros_like(c2_ref)

    rg_row = i * R + jax.lax.broadcasted_iota(jnp.int32, (1, R), 1)
    oh = ((rg_row >= lo_ref[...]) & (rg_row < hi_ref[...])).astype(
        jnp.bfloat16)
    rg_col = i * R + jax.lax.broadcasted_iota(jnp.int32, (R, 1), 0)
    x = jnp.where(rg_col < N, x_ref[...], 0.0)
    x2 = x * x
    xh = x.astype(jnp.bfloat16)
    xl = (x - xh.astype(jnp.float32)).astype(jnp.bfloat16)
    x2h = x2.astype(jnp.bfloat16)
    a_ref[...] += dot(oh, xh) + dot(oh, xl)
    c2_ref[...] += dot(oh, x2h)

    @pl.when(i == G - 1)
    def _():
        s = a_ref[...]
        mean = s * invc_ref[...]
        msm = mean * ms_ref[...]
        varsum = jnp.maximum(
            c2_ref[...] - 2.0 * msm * s + c_ref[...] * msm * msm, 0.0)
        a = w_ref[...] * jax.lax.rsqrt(varsum * invc_ref[...] + 1e-6)
        a_bf = a.astype(jnp.bfloat16).astype(jnp.float32)
        a_ref[...] = a_bf
        c2_ref[...] = b_ref[...] - a_bf * msm


def _apply_body(x_ref, lo_ref, hi_ref, a_ref, c2_ref, o_ref, *, R):
    i = pl.program_id(0)
    dot = functools.partial(jnp.dot, preferred_element_type=jnp.float32)
    rg_col = i * R + jax.lax.broadcasted_iota(jnp.int32, (R, 1), 0)
    oh = ((rg_col >= lo_ref[...]) & (rg_col < hi_ref[...])).astype(
        jnp.bfloat16)
    c = c2_ref[...]
    ah = a_ref[...].astype(jnp.bfloat16)
    ch = c.astype(jnp.bfloat16)
    cl = (c - ch.astype(jnp.float32)).astype(jnp.bfloat16)
    ar = dot(oh, ah)
    cr = dot(oh, ch) + dot(oh, cl)
    o_ref[...] = ar * x_ref[...] + cr


@jax.jit
def kernel(tensor, nodes_per_img, weight, bias, mean_scale):
    N, D = tensor.shape
    B = nodes_per_img.shape[0]
    R = 2048
    G = pl.cdiv(N, R)
    Bp = 384

    counts = nodes_per_img.astype(jnp.float32)
    sizes = nodes_per_img.astype(jnp.int32)
    hi = jnp.cumsum(sizes)
    lo = hi - sizes
    lo_p = jnp.full((Bp,), N, jnp.int32).at[:B].set(lo)
    hi_p = jnp.full((Bp,), N, jnp.int32).at[:B].set(hi)
    c_col = jnp.zeros((Bp, 1), jnp.float32).at[:B, 0].set(counts)
    invc_col = 1.0 / (c_col + 1e-6)

    def const(shape):
        return pl.BlockSpec(shape, lambda i: (0,) * len(shape))

    a_mat, c_mat = pl.pallas_call(
        functools.partial(_stats_body, R=R, N=N, G=G),
        grid=(G,),
        in_specs=[
            pl.BlockSpec((R, D), lambda i: (i, 0)),
            const((Bp, 1)), const((Bp, 1)),
            const((Bp, 1)), const((Bp, 1)),
            const((1, D)), const((1, D)), const((1, D)),
        ],
        out_specs=[const((Bp, D)), const((Bp, D))],
        out_shape=[jax.ShapeDtypeStruct((Bp, D), jnp.float32),
                   jax.ShapeDtypeStruct((Bp, D), jnp.float32)],
    )(
        tensor, lo_p.reshape(Bp, 1), hi_p.reshape(Bp, 1), c_col, invc_col,
        mean_scale.reshape(1, D), weight.reshape(1, D), bias.reshape(1, D),
    )

    out = pl.pallas_call(
        functools.partial(_apply_body, R=R),
        grid=(G,),
        in_specs=[
            pl.BlockSpec((R, D), lambda i: (i, 0)),
            const((1, Bp)), const((1, Bp)),
            const((Bp, D)), const((Bp, D)),
        ],
        out_specs=pl.BlockSpec((R, D), lambda i: (i, 0)),
        out_shape=jax.ShapeDtypeStruct((N, D), jnp.float32),
    )(tensor, lo_p.reshape(1, Bp), hi_p.reshape(1, Bp), a_mat, c_mat)
    return out
